# Initial kernel scaffold; baseline (speedup 1.0000x reference)
import functools

import jax
import jax.numpy as jnp
from jax import lax
from jax.experimental import pallas as pl
from jax.experimental.pallas import tpu as pltpu
from jax.experimental.pallas import tpu_sc as plsc

N = 10000
E = 320000
B = 64
BE = 1000
BN = 1000
GE = E // BE
GN = N // BN
HI = jax.lax.Precision.HIGHEST
F32 = jnp.float32
LOG2 = 0.6931471805599453


def _ssp(v):
    return jnp.maximum(v, 0.0) + jnp.log1p(jnp.exp(-jnp.abs(v))) - LOG2


def _c0(shape):
    return pl.BlockSpec(shape, lambda *_: tuple(0 for _ in shape))


def _pre_call(xv, uv, npre, spre):
    fin = xv.shape[1]

    def body(xv_ref, uv_ref, nw1, nb1, nw2, nb2, sw1, sb1, sw2, sb2,
             x0_ref, u0_ref):
        i = pl.program_id(0)
        t = _ssp(jnp.dot(xv_ref[...], nw1[...]) + nb1[...])
        x0_ref[...] = _ssp(jnp.dot(t, nw2[...]) + nb2[...])

        @pl.when(i == 0)
        def _():
            tu = _ssp(jnp.dot(uv_ref[...], sw1[...]) + sb1[...])
            u0_ref[...] = _ssp(jnp.dot(tu, sw2[...]) + sb2[...])

    args = [xv, uv,
            npre[0]["w"], npre[0]["b"].reshape(1, -1),
            npre[1]["w"], npre[1]["b"].reshape(1, -1),
            spre[0]["w"], spre[0]["b"].reshape(1, -1),
            spre[1]["w"], spre[1]["b"].reshape(1, -1)]
    in_specs = [pl.BlockSpec((BN, fin), lambda i: (i, 0))] + \
               [_c0(a.shape) for a in args[1:]]
    return pl.pallas_call(
        body,
        grid=(GN,),
        in_specs=in_specs,
        out_specs=[pl.BlockSpec((BN, 32), lambda i: (i, 0)), _c0((B, 32))],
        out_shape=[jax.ShapeDtypeStruct((N, 32), F32),
                   jax.ShapeDtypeStruct((B, 32), F32)],
    )(*args)


def _sc_gather(x0, src, dst):
    CH = 80
    PER = E // 32
    NCH = PER // CH
    mesh = plsc.VectorSubcoreMesh(core_axis_name="c", subcore_axis_name="s")

    @functools.partial(
        pl.kernel,
        out_type=(jax.ShapeDtypeStruct((E, 32), F32),
                  jax.ShapeDtypeStruct((E, 32), F32)),
        mesh=mesh,
        scratch_types=[pltpu.VMEM((CH,), jnp.int32),
                       pltpu.VMEM((CH,), jnp.int32),
                       pltpu.VMEM((CH, 32), F32),
                       pltpu.VMEM((CH, 32), F32),
                       pltpu.SemaphoreType.DMA,
                       pltpu.SemaphoreType.DMA],
    )
    def gk(x0_h, src_h, dst_h, gx_h, gy_h, ia, ib, ra, rb, s1, s2):
        wid = lax.axis_index("s") * 2 + lax.axis_index("c")
        base = wid * PER

        def body(j, carry):
            off = base + j * CH
            pltpu.sync_copy(src_h.at[pl.ds(off, CH)], ia)
            pltpu.sync_copy(dst_h.at[pl.ds(off, CH)], ib)
            c1 = pltpu.async_copy(x0_h.at[ia], ra, s1)
            c2 = pltpu.async_copy(x0_h.at[ib], rb, s2)
            c1.wait()
            c2.wait()
            pltpu.sync_copy(ra, gx_h.at[pl.ds(off, CH)])
            pltpu.sync_copy(rb, gy_h.at[pl.ds(off, CH)])
            return carry

        lax.fori_loop(0, NCH, body, 0)

    return gk(x0, src, dst)


def _sc_scatter(e1, dst, with_counts):
    CH = 80
    PER = E // 32
    NCH = PER // CH
    ZCH = 400
    NZ = N // ZCH
    mesh = plsc.VectorSubcoreMesh(core_axis_name="c", subcore_axis_name="s")

    outs = [jax.ShapeDtypeStruct((2 * N, 32), F32)]
    scr = [pltpu.VMEM_SHARED((N, 32), F32),
           pltpu.VMEM((CH,), jnp.int32),
           pltpu.VMEM((CH, 32), F32)]
    if with_counts:
        outs.append(jax.ShapeDtypeStruct((2 * N, 32), F32))
        scr.insert(1, pltpu.VMEM_SHARED((N, 32), F32))
        scr.append(pltpu.VMEM((CH, 32), F32))

    zrows = jnp.zeros((ZCH, 32), F32)
    ones = jnp.ones((CH, 32), F32)

    @functools.partial(pl.kernel, out_type=tuple(outs), mesh=mesh,
                       scratch_types=scr)
    def sk(*refs):
        if with_counts:
            (e1_h, dst_h, z_h, on_h, ps_h, pc_h,
             acc, accc, idx, rows, ones_v) = refs
        else:
            e1_h, dst_h, z_h, on_h, ps_h, acc, idx, rows = refs
        cid = lax.axis_index("c")
        sid = lax.axis_index("s")
        wid = sid * 2 + cid
        base = wid * PER

        pltpu.sync_copy(z_h, acc.at[pl.ds(sid * ZCH, ZCH)])
        if with_counts:
            pltpu.sync_copy(z_h, accc.at[pl.ds(sid * ZCH, ZCH)])
            pltpu.sync_copy(on_h, ones_v)

        @pl.when(sid < NZ - 16)
        def _():
            pltpu.sync_copy(z_h, acc.at[pl.ds((sid + 16) * ZCH, ZCH)])
            if with_counts:
                pltpu.sync_copy(z_h, accc.at[pl.ds((sid + 16) * ZCH, ZCH)])

        plsc.subcore_barrier()

        def body(j, carry):
            off = base + j * CH
            pltpu.sync_copy(dst_h.at[pl.ds(off, CH)], idx)
            pltpu.sync_copy(e1_h.at[pl.ds(off, CH)], rows)
            pltpu.sync_copy(rows, acc.at[idx], add=True)
            if with_counts:
                pltpu.sync_copy(ones_v, accc.at[idx], add=True)
            return carry

        lax.fori_loop(0, NCH, body, 0)
        plsc.subcore_barrier()

        pltpu.sync_copy(acc.at[pl.ds(sid * ZCH, ZCH)],
                        ps_h.at[pl.ds(cid * N + sid * ZCH, ZCH)])
        if with_counts:
            pltpu.sync_copy(accc.at[pl.ds(sid * ZCH, ZCH)],
                            pc_h.at[pl.ds(cid * N + sid * ZCH, ZCH)])

        @pl.when(sid < NZ - 16)
        def _():
            pltpu.sync_copy(acc.at[pl.ds((sid + 16) * ZCH, ZCH)],
                            ps_h.at[pl.ds(cid * N + (sid + 16) * ZCH, ZCH)])
            if with_counts:
                pltpu.sync_copy(accc.at[pl.ds((sid + 16) * ZCH, ZCH)],
                                pc_h.at[pl.ds(cid * N + (sid + 16) * ZCH,
                                              ZCH)])

    res = sk(e1, dst, zrows, ones)
    if with_counts:
        ps, pc = res
        return ps.reshape(2, N, 32), pc.reshape(2, N, 32)
    return res.reshape(2, N, 32)


def _edge_call(ev, gx, gy, bbrow, bbcol, u0, p, first):
    fe = ev.shape[1]
    phe = p["phi_e"]
    epre = p["epre"]

    def body(ev_ref, gx_ref, gy_ref, br_ref, bc_ref, u0_ref,
             ew1, eb1, ew2, eb2, w1, b1, w2, b2, w3, b3,
             e1_ref, ue_ref, ce_ref):
        i = pl.program_id(0)
        e0 = _ssp(jnp.dot(ev_ref[...], ew1[...]) + eb1[...])
        e0 = _ssp(jnp.dot(e0, ew2[...]) + eb2[...])
        bc = bc_ref[0]
        mask_e = (lax.broadcasted_iota(F32, (BE, B), 1) == bc).astype(F32)
        ubb = jnp.dot(mask_e, u0_ref[...])
        cc = jnp.concatenate([gx_ref[...], gy_ref[...], e0, ubb], axis=1)
        h = _ssp(jnp.dot(cc, w1[...]) + b1[...])
        h = _ssp(jnp.dot(h, w2[...]) + b2[...])
        e1 = jnp.dot(h, w3[...]) + b3[...] + (e0 if first else ev_ref[...])
        e1_ref[...] = e1
        br = br_ref[0]
        mask_t = (lax.broadcasted_iota(F32, (B, BE), 0) == br).astype(F32)

        @pl.when(i == 0)
        def _():
            ue_ref[...] = jnp.zeros_like(ue_ref)
            ce_ref[...] = jnp.zeros_like(ce_ref)

        ue_ref[...] += jnp.dot(mask_t, e1, precision=HI)
        ce_ref[...] += jnp.broadcast_to(
            jnp.sum(mask_t, axis=1, keepdims=True), (B, 32))

    args = [ev, gx, gy, bbrow, bbcol, u0,
            epre[0]["w"], epre[0]["b"].reshape(1, -1),
            epre[1]["w"], epre[1]["b"].reshape(1, -1),
            phe[0]["w"], phe[0]["b"].reshape(1, -1),
            phe[1]["w"], phe[1]["b"].reshape(1, -1),
            phe[2]["w"], phe[2]["b"].reshape(1, -1)]
    in_specs = [pl.BlockSpec((BE, fe), lambda i: (i, 0)),
                pl.BlockSpec((BE, 32), lambda i: (i, 0)),
                pl.BlockSpec((BE, 32), lambda i: (i, 0)),
                pl.BlockSpec((1, 1, BE), lambda i: (i, 0, 0)),
                pl.BlockSpec((1, BE, 1), lambda i: (i, 0, 0)),
                _c0((B, 32))] + [_c0(a.shape) for a in args[6:]]
    return pl.pallas_call(
        body,
        grid=(GE,),
        in_specs=in_specs,
        out_specs=[pl.BlockSpec((BE, 32), lambda i: (i, 0)),
                   _c0((B, 32)), _c0((B, 32))],
        out_shape=[jax.ShapeDtypeStruct((E, 32), F32),
                   jax.ShapeDtypeStruct((B, 32), F32),
                   jax.ShapeDtypeStruct((B, 32), F32)],
    )(*args)


def _node_call(psum, cnt_in, x0, skip, brow, bcol, u0, usk, ue_sum, cnt_e,
               p, first):
    phv = p["phi_v"]
    phu = p["phi_u"]

    def body(ps_ref, c_ref, x0_ref, sk_ref, br_ref, bc_ref, u0_ref, us_ref,
             ues_ref, ce_ref, vw1, vb1, vw2, vb2, vw3, vb3,
             uw1, ub1, uw2, ub2, uw3, ub3,
             x1_ref, u1_ref, uxm_ref, cn_ref, *rest):
        i = pl.program_id(0)
        s = ps_ref[0] + ps_ref[1]
        if first:
            c = c_ref[0] + c_ref[1]
            rest[0][...] = c
        else:
            c = c_ref[...]
        agg = s / jnp.maximum(c, 1.0)
        bc = bc_ref[0]
        mask_n = (lax.broadcasted_iota(F32, (BN, B), 1) == bc).astype(F32)
        ub = jnp.dot(mask_n, u0_ref[...])
        cc = jnp.concatenate([agg, x0_ref[...], ub], axis=1)
        t = _ssp(jnp.dot(cc, vw1[...]) + vb1[...])
        t = _ssp(jnp.dot(t, vw2[...]) + vb2[...])
        x1 = jnp.dot(t, vw3[...]) + vb3[...] + sk_ref[...]
        x1_ref[...] = x1
        br = br_ref[0]
        mask_t = (lax.broadcasted_iota(F32, (B, BN), 0) == br).astype(F32)

        @pl.when(i == 0)
        def _():
            uxm_ref[...] = jnp.zeros_like(uxm_ref)
            cn_ref[...] = jnp.zeros_like(cn_ref)

        uxm_ref[...] += jnp.dot(mask_t, x1, precision=HI)
        cn_ref[...] += jnp.broadcast_to(
            jnp.sum(mask_t, axis=1, keepdims=True), (B, 32))

        @pl.when(i == GN - 1)
        def _():
            ux = uxm_ref[...] / jnp.maximum(cn_ref[...], 1.0)
            ue = ues_ref[...] / jnp.maximum(ce_ref[...], 1.0)
            ccu = jnp.concatenate([ue, ux, u0_ref[...]], axis=1)
            tu = _ssp(jnp.dot(ccu, uw1[...]) + ub1[...])
            tu = _ssp(jnp.dot(tu, uw2[...]) + ub2[...])
            u1_ref[...] = jnp.dot(tu, uw3[...]) + ub3[...] + us_ref[...]
            uxm_ref[...] = ux

    args = [psum, cnt_in, x0, skip, brow, bcol, u0, usk, ue_sum, cnt_e,
            phv[0]["w"], phv[0]["b"].reshape(1, -1),
            phv[1]["w"], phv[1]["b"].reshape(1, -1),
            phv[2]["w"], phv[2]["b"].reshape(1, -1),
            phu[0]["w"], phu[0]["b"].reshape(1, -1),
            phu[1]["w"], phu[1]["b"].reshape(1, -1),
            phu[2]["w"], phu[2]["b"].reshape(1, -1)]
    cnt_spec = (pl.BlockSpec((2, BN, 32), lambda i: (0, i, 0)) if first
                else pl.BlockSpec((BN, 32), lambda i: (i, 0)))
    in_specs = [pl.BlockSpec((2, BN, 32), lambda i: (0, i, 0)),
                cnt_spec,
                pl.BlockSpec((BN, 32), lambda i: (i, 0)),
                pl.BlockSpec((BN, 32), lambda i: (i, 0)),
                pl.BlockSpec((1, 1, BN), lambda i: (i, 0, 0)),
                pl.BlockSpec((1, BN, 1), lambda i: (i, 0, 0)),
                _c0((B, 32)), _c0((B, 32)), _c0((B, 32)), _c0((B, 32))] + \
               [_c0(a.shape) for a in args[10:]]
    out_specs = [pl.BlockSpec((BN, 32), lambda i: (i, 0)),
                 _c0((B, 32)), _c0((B, 32)), _c0((B, 32))]
    out_shape = [jax.ShapeDtypeStruct((N, 32), F32),
                 jax.ShapeDtypeStruct((B, 32), F32),
                 jax.ShapeDtypeStruct((B, 32), F32),
                 jax.ShapeDtypeStruct((B, 32), F32)]
    if first:
        out_specs.append(pl.BlockSpec((BN, 32), lambda i: (i, 0)))
        out_shape.append(jax.ShapeDtypeStruct((N, 32), F32))
    return pl.pallas_call(
        body,
        grid=(GN,),
        in_specs=in_specs,
        out_specs=out_specs,
        out_shape=out_shape,
    )(*args)


def _head_rnn_call(uxm, uem, uv, params):
    pp = params["preproc"]
    hl = params["hiddens"]
    rp = params["rnn"]

    def body(uxm_ref, uem_ref, uv_ref, pw, pb, h1, hb1, h2, hb2, h3, hb3,
             wh, bh, wo, bo, su_ref, sd_ref, f_ref):
        zz = jnp.zeros((B, 32), F32)
        cc = jnp.concatenate([zz, uxm_ref[...], zz, uem_ref[...],
                              uv_ref[...]], axis=1)
        tmp = jnp.dot(cc, pw[...]) + pb[...]
        cf = jnp.concatenate([tmp, jnp.zeros_like(tmp)], axis=1)
        f = _ssp(jnp.dot(cf, h1[...]) + hb1[...])
        f = _ssp(jnp.dot(f, h2[...]) + hb2[...])
        f_ref[...] = jnp.dot(f, h3[...]) + hb3[...]
        h = tmp
        for t in range(10):
            h = jnp.tanh(jnp.dot(h, wh[...]) + bh[...])
            o = jnp.dot(h, wo[...]) + bo[...]
            su_ref[:, (9 - t) * 128:(10 - t) * 128] = o[:, 256:384]
            su_ref[:, (10 + t) * 128:(11 + t) * 128] = o[:, 0:128]
            sd_ref[:, (9 - t) * 128:(10 - t) * 128] = o[:, 384:512]
            sd_ref[:, (10 + t) * 128:(11 + t) * 128] = o[:, 128:256]

    args = [uxm, uem, uv,
            pp["w"], pp["b"].reshape(1, -1),
            hl[0]["w"], hl[0]["b"].reshape(1, -1),
            hl[1]["w"], hl[1]["b"].reshape(1, -1),
            hl[2]["w"], hl[2]["b"].reshape(1, -1),
            rp["wh"], rp["bh"].reshape(1, -1),
            rp["wo"], rp["bo"].reshape(1, -1)]
    return pl.pallas_call(
        body,
        in_specs=[_c0(a.shape) for a in args],
        out_specs=[_c0((B, 20 * 128)), _c0((B, 20 * 128)), _c0((B, 1))],
        out_shape=[jax.ShapeDtypeStruct((B, 20 * 128), F32),
                   jax.ShapeDtypeStruct((B, 20 * 128), F32),
                   jax.ShapeDtypeStruct((B, 1), F32)],
    )(*args)


def _attn_call(su3, sd3, params):
    ap = params["attn"]
    hl = params["hiddens"]

    def body(su_ref, sd_ref, wq, wk, wv, h1, hb1, h2, hb2, h3, hb3,
             up_ref, dn_ref):
        su = su_ref[0]
        sd = sd_ref[0]

        def cross(a, b):
            q = jnp.dot(a, wq[...])
            k = jnp.dot(b, wk[...])
            v = jnp.dot(b, wv[...])
            s = lax.dot_general(q, k, (((1,), (1,)), ((), ()))) * \
                (1.0 / jnp.sqrt(128.0))
            s = s - jnp.max(s, axis=-1, keepdims=True)
            e = jnp.exp(s)
            w = e / jnp.sum(e, axis=-1, keepdims=True)
            return a + jnp.dot(w, v)

        def hid(v):
            hh = _ssp(jnp.dot(v, h1[...]) + hb1[...])
            hh = _ssp(jnp.dot(hh, h2[...]) + hb2[...])
            return jnp.dot(hh, h3[...]) + hb3[...]

        up_ref[0] = hid(cross(su, sd))
        dn_ref[0] = hid(cross(sd, su))

    args = [su3, sd3, ap["wq"], ap["wk"], ap["wv"],
            hl[0]["w"], hl[0]["b"].reshape(1, -1),
            hl[1]["w"], hl[1]["b"].reshape(1, -1),
            hl[2]["w"], hl[2]["b"].reshape(1, -1)]
    in_specs = [pl.BlockSpec((1, 20, 128), lambda g: (g, 0, 0)),
                pl.BlockSpec((1, 20, 128), lambda g: (g, 0, 0))] + \
               [_c0(a.shape) for a in args[2:]]
    return pl.pallas_call(
        body,
        grid=(B,),
        in_specs=in_specs,
        out_specs=[pl.BlockSpec((1, 20, 1), lambda g: (g, 0, 0)),
                   pl.BlockSpec((1, 20, 1), lambda g: (g, 0, 0))],
        out_shape=[jax.ShapeDtypeStruct((B, 20, 1), F32),
                   jax.ShapeDtypeStruct((B, 20, 1), F32)],
    )(*args)


def kernel(x, edge_index, edge_attr, state, batch, bond_batch, params):
    src = edge_index[0]
    dst = edge_index[1]
    bbrow = bond_batch.astype(F32).reshape(GE, 1, BE)
    bbcol = bond_batch.astype(F32).reshape(GE, BE, 1)
    brow = batch.astype(F32).reshape(GN, 1, BN)
    bcol = batch.astype(F32).reshape(GN, BN, 1)

    xv = x.astype(F32)
    ev = edge_attr
    uv = state
    cnt_n32 = None
    uxm = uem = None
    for bi, p in enumerate(params["blocks"]):
        first = bi == 0
        x0, u0 = _pre_call(xv, uv, p["npre"], p["spre"])
        gx, gy = _sc_gather(x0, src, dst)
        e1, ue_sum, cnt_e = _edge_call(ev, gx, gy, bbrow, bbcol, u0, p, first)
        if first:
            psum, pcnt = _sc_scatter(e1, dst, True)
            x1, u1, uxm, _, cnt_n32 = _node_call(
                psum, pcnt, x0, x0, brow, bcol, u0, u0, ue_sum, cnt_e,
                p, True)
        else:
            psum = _sc_scatter(e1, dst, False)
            x1, u1, uxm, _ = _node_call(
                psum, cnt_n32, x0, xv, brow, bcol, u0, uv, ue_sum, cnt_e,
                p, False)
        uem = ue_sum / jnp.maximum(cnt_e, 1.0)
        xv, ev, uv = x1, e1, u1

    su_w, sd_w, fermi = _head_rnn_call(uxm, uem, uv, params)
    up, dn = _attn_call(su_w.reshape(B, 20, 128), sd_w.reshape(B, 20, 128),
                        params)
    hu = up[..., 0]
    hd = dn[..., 0]
    x_uu = hu[:, 10:]
    x_ud = hd[:, 10:]
    x_du = jnp.flip(hu[:, :10], axis=1)
    x_dd = jnp.flip(hd[:, :10], axis=1)
    return (x_uu, x_ud, x_du, x_dd, fermi)


# speedup vs baseline: 2.5583x; 2.5583x over previous
import functools

import jax
import jax.numpy as jnp
from jax import lax
from jax.experimental import pallas as pl
from jax.experimental.pallas import tpu as pltpu
from jax.experimental.pallas import tpu_sc as plsc

N = 10000
E = 320000
B = 64
BE = 1000
BN = 1000
GE = E // BE
GN = N // BN
HI = jax.lax.Precision.HIGHEST
F32 = jnp.float32
LOG2 = 0.6931471805599453


def _dot(a, b):
    return jnp.dot(a.astype(jnp.bfloat16), b.astype(jnp.bfloat16),
                   preferred_element_type=jnp.float32)


def _dot_hi(a, b):
    return jnp.dot(a, b, precision=HI)


def _ssp(v):
    return jnp.maximum(v, 0.0) + jnp.log1p(jnp.exp(-jnp.abs(v))) - LOG2


def _c0(shape):
    return pl.BlockSpec(shape, lambda *_: tuple(0 for _ in shape))


def _pre_call(xv, uv, npre, spre):
    fin = xv.shape[1]

    def body(xv_ref, uv_ref, nw1, nb1, nw2, nb2, sw1, sb1, sw2, sb2,
             x0_ref, u0_ref):
        i = pl.program_id(0)
        t = _ssp(_dot(xv_ref[...], nw1[...]) + nb1[...])
        x0_ref[...] = _ssp(_dot(t, nw2[...]) + nb2[...])

        @pl.when(i == 0)
        def _():
            tu = _ssp(_dot(uv_ref[...], sw1[...]) + sb1[...])
            u0_ref[...] = _ssp(_dot(tu, sw2[...]) + sb2[...])

    args = [xv, uv,
            npre[0]["w"], npre[0]["b"].reshape(1, -1),
            npre[1]["w"], npre[1]["b"].reshape(1, -1),
            spre[0]["w"], spre[0]["b"].reshape(1, -1),
            spre[1]["w"], spre[1]["b"].reshape(1, -1)]
    in_specs = [pl.BlockSpec((BN, fin), lambda i: (i, 0))] + \
               [_c0(a.shape) for a in args[1:]]
    return pl.pallas_call(
        body,
        grid=(GN,),
        in_specs=in_specs,
        out_specs=[pl.BlockSpec((BN, 32), lambda i: (i, 0)), _c0((B, 32))],
        out_shape=[jax.ShapeDtypeStruct((N, 32), F32),
                   jax.ShapeDtypeStruct((B, 32), F32)],
    )(*args)


def _sc_gather(x0, src, dst):
    CH = 80
    PER = E // 32
    NCH = PER // CH
    mesh = plsc.VectorSubcoreMesh(core_axis_name="c", subcore_axis_name="s")

    @functools.partial(
        pl.kernel,
        out_type=(jax.ShapeDtypeStruct((E, 32), F32),
                  jax.ShapeDtypeStruct((E, 32), F32)),
        mesh=mesh,
        compiler_params=pltpu.CompilerParams(use_tc_tiling_on_sc=False),
        scratch_types=[pltpu.VMEM((CH,), jnp.int32),
                       pltpu.VMEM((CH,), jnp.int32),
                       pltpu.VMEM((CH, 32), F32),
                       pltpu.VMEM((CH, 32), F32),
                       pltpu.SemaphoreType.DMA,
                       pltpu.SemaphoreType.DMA],
    )
    def gk(x0_h, src_h, dst_h, gx_h, gy_h, ia, ib, ra, rb, s1, s2):
        wid = lax.axis_index("s") * 2 + lax.axis_index("c")
        base = wid * PER

        def body(j, carry):
            off = base + j * CH
            pltpu.sync_copy(src_h.at[pl.ds(off, CH)], ia)
            pltpu.sync_copy(dst_h.at[pl.ds(off, CH)], ib)
            c1 = pltpu.async_copy(x0_h.at[ia], ra, s1)
            c2 = pltpu.async_copy(x0_h.at[ib], rb, s2)
            c1.wait()
            c2.wait()
            pltpu.sync_copy(ra, gx_h.at[pl.ds(off, CH)])
            pltpu.sync_copy(rb, gy_h.at[pl.ds(off, CH)])
            return carry

        lax.fori_loop(0, NCH, body, 0)

    return gk(x0, src, dst)


def _sc_scatter(e1, dst, with_counts):
    CH = 80
    PER = E // 32
    NCH = PER // CH
    ZCH = 400
    NZ = N // ZCH
    mesh = plsc.VectorSubcoreMesh(core_axis_name="c", subcore_axis_name="s")

    outs = [jax.ShapeDtypeStruct((2 * N, 32), F32)]
    scr = [pltpu.VMEM_SHARED((N, 32), F32),
           pltpu.VMEM((CH,), jnp.int32),
           pltpu.VMEM((CH, 32), F32)]
    if with_counts:
        outs.append(jax.ShapeDtypeStruct((2 * N, 32), F32))
        scr.insert(1, pltpu.VMEM_SHARED((N, 32), F32))
        scr.append(pltpu.VMEM((CH, 32), F32))

    zrows = jnp.zeros((ZCH, 32), F32)
    ones = jnp.ones((CH, 32), F32)

    @functools.partial(
        pl.kernel, out_type=tuple(outs), mesh=mesh,
        compiler_params=pltpu.CompilerParams(use_tc_tiling_on_sc=False),
        scratch_types=scr)
    def sk(*refs):
        if with_counts:
            (e1_h, dst_h, z_h, on_h, ps_h, pc_h,
             acc, accc, idx, rows, ones_v) = refs
        else:
            e1_h, dst_h, z_h, on_h, ps_h, acc, idx, rows = refs
        cid = lax.axis_index("c")
        sid = lax.axis_index("s")
        wid = sid * 2 + cid
        base = wid * PER

        pltpu.sync_copy(z_h, acc.at[pl.ds(sid * ZCH, ZCH)])
        if with_counts:
            pltpu.sync_copy(z_h, accc.at[pl.ds(sid * ZCH, ZCH)])
            pltpu.sync_copy(on_h, ones_v)

        @pl.when(sid < NZ - 16)
        def _():
            pltpu.sync_copy(z_h, acc.at[pl.ds((sid + 16) * ZCH, ZCH)])
            if with_counts:
                pltpu.sync_copy(z_h, accc.at[pl.ds((sid + 16) * ZCH, ZCH)])

        plsc.subcore_barrier()

        def body(j, carry):
            off = base + j * CH
            pltpu.sync_copy(dst_h.at[pl.ds(off, CH)], idx)
            pltpu.sync_copy(e1_h.at[pl.ds(off, CH)], rows)
            pltpu.sync_copy(rows, acc.at[idx], add=True)
            if with_counts:
                pltpu.sync_copy(ones_v, accc.at[idx], add=True)
            return carry

        lax.fori_loop(0, NCH, body, 0)
        plsc.subcore_barrier()

        pltpu.sync_copy(acc.at[pl.ds(sid * ZCH, ZCH)],
                        ps_h.at[pl.ds(cid * N + sid * ZCH, ZCH)])
        if with_counts:
            pltpu.sync_copy(accc.at[pl.ds(sid * ZCH, ZCH)],
                            pc_h.at[pl.ds(cid * N + sid * ZCH, ZCH)])

        @pl.when(sid < NZ - 16)
        def _():
            pltpu.sync_copy(acc.at[pl.ds((sid + 16) * ZCH, ZCH)],
                            ps_h.at[pl.ds(cid * N + (sid + 16) * ZCH, ZCH)])
            if with_counts:
                pltpu.sync_copy(accc.at[pl.ds((sid + 16) * ZCH, ZCH)],
                                pc_h.at[pl.ds(cid * N + (sid + 16) * ZCH,
                                              ZCH)])

    res = sk(e1, dst, zrows, ones)
    if with_counts:
        ps, pc = res
        return ps.reshape(2, N, 32), pc.reshape(2, N, 32)
    if isinstance(res, (tuple, list)):
        res = res[0]
    return res.reshape(2, N, 32)


def _edge_call(ev, gx, gy, bbrow, bbcol, u0, p, first):
    fe = ev.shape[1]
    phe = p["phi_e"]
    epre = p["epre"]

    def body(ev_ref, gx_ref, gy_ref, br_ref, bc_ref, u0_ref,
             ew1, eb1, ew2, eb2, w1, b1, w2, b2, w3, b3,
             e1_ref, ue_ref, ce_ref):
        i = pl.program_id(0)
        e0 = _ssp(_dot(ev_ref[...], ew1[...]) + eb1[...])
        e0 = _ssp(_dot(e0, ew2[...]) + eb2[...])
        bc = bc_ref[0]
        mask_e = (lax.broadcasted_iota(jnp.int32, (BE, B), 1).astype(F32) == bc).astype(F32)
        ubb = _dot(mask_e, u0_ref[...])
        cc = jnp.concatenate([gx_ref[...], gy_ref[...], e0, ubb], axis=1)
        h = _ssp(_dot(cc, w1[...]) + b1[...])
        h = _ssp(_dot(h, w2[...]) + b2[...])
        e1 = _dot(h, w3[...]) + b3[...] + (e0 if first else ev_ref[...])
        e1_ref[...] = e1
        br = br_ref[0]
        mask_t = (lax.broadcasted_iota(jnp.int32, (B, BE), 0).astype(F32) == br).astype(F32)

        @pl.when(i == 0)
        def _():
            ue_ref[...] = jnp.zeros_like(ue_ref)
            ce_ref[...] = jnp.zeros_like(ce_ref)

        ue_ref[...] += _dot_hi(mask_t, e1)
        ce_ref[...] += jnp.broadcast_to(
            jnp.sum(mask_t, axis=1, keepdims=True), (B, 32))

    args = [ev, gx, gy, bbrow, bbcol, u0,
            epre[0]["w"], epre[0]["b"].reshape(1, -1),
            epre[1]["w"], epre[1]["b"].reshape(1, -1),
            phe[0]["w"], phe[0]["b"].reshape(1, -1),
            phe[1]["w"], phe[1]["b"].reshape(1, -1),
            phe[2]["w"], phe[2]["b"].reshape(1, -1)]
    in_specs = [pl.BlockSpec((BE, fe), lambda i: (i, 0)),
                pl.BlockSpec((BE, 32), lambda i: (i, 0)),
                pl.BlockSpec((BE, 32), lambda i: (i, 0)),
                pl.BlockSpec((1, 1, BE), lambda i: (i, 0, 0)),
                pl.BlockSpec((1, BE, 1), lambda i: (i, 0, 0)),
                _c0((B, 32))] + [_c0(a.shape) for a in args[6:]]
    return pl.pallas_call(
        body,
        grid=(GE,),
        in_specs=in_specs,
        out_specs=[pl.BlockSpec((BE, 32), lambda i: (i, 0)),
                   _c0((B, 32)), _c0((B, 32))],
        out_shape=[jax.ShapeDtypeStruct((E, 32), F32),
                   jax.ShapeDtypeStruct((B, 32), F32),
                   jax.ShapeDtypeStruct((B, 32), F32)],
    )(*args)


def _node_call(psum, cnt_in, x0, skip, brow, bcol, u0, usk, ue_sum, cnt_e,
               p, first):
    phv = p["phi_v"]
    phu = p["phi_u"]

    def body(ps_ref, c_ref, x0_ref, sk_ref, br_ref, bc_ref, u0_ref, us_ref,
             ues_ref, ce_ref, vw1, vb1, vw2, vb2, vw3, vb3,
             uw1, ub1, uw2, ub2, uw3, ub3,
             x1_ref, u1_ref, uxm_ref, cn_ref, *rest):
        i = pl.program_id(0)
        s = ps_ref[0] + ps_ref[1]
        if first:
            c = c_ref[0] + c_ref[1]
            rest[0][...] = c
        else:
            c = c_ref[...]
        agg = s / jnp.maximum(c, 1.0)
        bc = bc_ref[0]
        mask_n = (lax.broadcasted_iota(jnp.int32, (BN, B), 1).astype(F32) == bc).astype(F32)
        ub = _dot(mask_n, u0_ref[...])
        cc = jnp.concatenate([agg, x0_ref[...], ub], axis=1)
        t = _ssp(_dot(cc, vw1[...]) + vb1[...])
        t = _ssp(_dot(t, vw2[...]) + vb2[...])
        x1 = _dot(t, vw3[...]) + vb3[...] + sk_ref[...]
        x1_ref[...] = x1
        br = br_ref[0]
        mask_t = (lax.broadcasted_iota(jnp.int32, (B, BN), 0).astype(F32) == br).astype(F32)

        @pl.when(i == 0)
        def _():
            uxm_ref[...] = jnp.zeros_like(uxm_ref)
            cn_ref[...] = jnp.zeros_like(cn_ref)

        uxm_ref[...] += _dot_hi(mask_t, x1)
        cn_ref[...] += jnp.broadcast_to(
            jnp.sum(mask_t, axis=1, keepdims=True), (B, 32))

        @pl.when(i == GN - 1)
        def _():
            ux = uxm_ref[...] / jnp.maximum(cn_ref[...], 1.0)
            ue = ues_ref[...] / jnp.maximum(ce_ref[...], 1.0)
            ccu = jnp.concatenate([ue, ux, u0_ref[...]], axis=1)
            tu = _ssp(_dot(ccu, uw1[...]) + ub1[...])
            tu = _ssp(_dot(tu, uw2[...]) + ub2[...])
            u1_ref[...] = _dot(tu, uw3[...]) + ub3[...] + us_ref[...]
            uxm_ref[...] = ux

    args = [psum, cnt_in, x0, skip, brow, bcol, u0, usk, ue_sum, cnt_e,
            phv[0]["w"], phv[0]["b"].reshape(1, -1),
            phv[1]["w"], phv[1]["b"].reshape(1, -1),
            phv[2]["w"], phv[2]["b"].reshape(1, -1),
            phu[0]["w"], phu[0]["b"].reshape(1, -1),
            phu[1]["w"], phu[1]["b"].reshape(1, -1),
            phu[2]["w"], phu[2]["b"].reshape(1, -1)]
    cnt_spec = (pl.BlockSpec((2, BN, 32), lambda i: (0, i, 0)) if first
                else pl.BlockSpec((BN, 32), lambda i: (i, 0)))
    in_specs = [pl.BlockSpec((2, BN, 32), lambda i: (0, i, 0)),
                cnt_spec,
                pl.BlockSpec((BN, 32), lambda i: (i, 0)),
                pl.BlockSpec((BN, 32), lambda i: (i, 0)),
                pl.BlockSpec((1, 1, BN), lambda i: (i, 0, 0)),
                pl.BlockSpec((1, BN, 1), lambda i: (i, 0, 0)),
                _c0((B, 32)), _c0((B, 32)), _c0((B, 32)), _c0((B, 32))] + \
               [_c0(a.shape) for a in args[10:]]
    out_specs = [pl.BlockSpec((BN, 32), lambda i: (i, 0)),
                 _c0((B, 32)), _c0((B, 32)), _c0((B, 32))]
    out_shape = [jax.ShapeDtypeStruct((N, 32), F32),
                 jax.ShapeDtypeStruct((B, 32), F32),
                 jax.ShapeDtypeStruct((B, 32), F32),
                 jax.ShapeDtypeStruct((B, 32), F32)]
    if first:
        out_specs.append(pl.BlockSpec((BN, 32), lambda i: (i, 0)))
        out_shape.append(jax.ShapeDtypeStruct((N, 32), F32))
    return pl.pallas_call(
        body,
        grid=(GN,),
        in_specs=in_specs,
        out_specs=out_specs,
        out_shape=out_shape,
    )(*args)


def _head_rnn_call(uxm, uem, uv, params):
    pp = params["preproc"]
    hl = params["hiddens"]
    rp = params["rnn"]

    def body(uxm_ref, uem_ref, uv_ref, pw, pb, h1, hb1, h2, hb2, h3, hb3,
             wh, bh, wo, bo, su_ref, sd_ref, f_ref):
        zz = jnp.zeros((B, 32), F32)
        cc = jnp.concatenate([zz, uxm_ref[...], zz, uem_ref[...],
                              uv_ref[...]], axis=1)
        tmp = _dot(cc, pw[...]) + pb[...]
        cf = jnp.concatenate([tmp, jnp.zeros_like(tmp)], axis=1)
        f = _ssp(_dot(cf, h1[...]) + hb1[...])
        f = _ssp(_dot(f, h2[...]) + hb2[...])
        f_ref[...] = _dot(f, h3[...]) + hb3[...]
        h = tmp
        for t in range(10):
            h = jnp.tanh(_dot(h, wh[...]) + bh[...])
            o = _dot(h, wo[...]) + bo[...]
            su_ref[:, (9 - t) * 128:(10 - t) * 128] = o[:, 256:384]
            su_ref[:, (10 + t) * 128:(11 + t) * 128] = o[:, 0:128]
            sd_ref[:, (9 - t) * 128:(10 - t) * 128] = o[:, 384:512]
            sd_ref[:, (10 + t) * 128:(11 + t) * 128] = o[:, 128:256]

    args = [uxm, uem, uv,
            pp["w"], pp["b"].reshape(1, -1),
            hl[0]["w"], hl[0]["b"].reshape(1, -1),
            hl[1]["w"], hl[1]["b"].reshape(1, -1),
            hl[2]["w"], hl[2]["b"].reshape(1, -1),
            rp["wh"], rp["bh"].reshape(1, -1),
            rp["wo"], rp["bo"].reshape(1, -1)]
    return pl.pallas_call(
        body,
        in_specs=[_c0(a.shape) for a in args],
        out_specs=[_c0((B, 20 * 128)), _c0((B, 20 * 128)), _c0((B, 1))],
        out_shape=[jax.ShapeDtypeStruct((B, 20 * 128), F32),
                   jax.ShapeDtypeStruct((B, 20 * 128), F32),
                   jax.ShapeDtypeStruct((B, 1), F32)],
    )(*args)


def _attn_call(su3, sd3, params):
    ap = params["attn"]
    hl = params["hiddens"]

    def body(su_ref, sd_ref, wq, wk, wv, h1, hb1, h2, hb2, h3, hb3,
             up_ref, dn_ref):
        su = su_ref[0]
        sd = sd_ref[0]

        def cross(a, b):
            q = _dot(a, wq[...])
            k = _dot(b, wk[...])
            v = _dot(b, wv[...])
            s = lax.dot_general(q.astype(jnp.bfloat16), k.astype(jnp.bfloat16),
                                (((1,), (1,)), ((), ())),
                                preferred_element_type=jnp.float32) * \
                (1.0 / jnp.sqrt(128.0))
            s = s - jnp.max(s, axis=-1, keepdims=True)
            e = jnp.exp(s)
            w = e / jnp.sum(e, axis=-1, keepdims=True)
            return a + _dot(w, v)

        def hid(v):
            hh = _ssp(_dot(v, h1[...]) + hb1[...])
            hh = _ssp(_dot(hh, h2[...]) + hb2[...])
            return _dot(hh, h3[...]) + hb3[...]

        up_ref[0] = hid(cross(su, sd))
        dn_ref[0] = hid(cross(sd, su))

    args = [su3, sd3, ap["wq"], ap["wk"], ap["wv"],
            hl[0]["w"], hl[0]["b"].reshape(1, -1),
            hl[1]["w"], hl[1]["b"].reshape(1, -1),
            hl[2]["w"], hl[2]["b"].reshape(1, -1)]
    in_specs = [pl.BlockSpec((1, 20, 128), lambda g: (g, 0, 0)),
                pl.BlockSpec((1, 20, 128), lambda g: (g, 0, 0))] + \
               [_c0(a.shape) for a in args[2:]]
    return pl.pallas_call(
        body,
        grid=(B,),
        in_specs=in_specs,
        out_specs=[pl.BlockSpec((1, 20, 1), lambda g: (g, 0, 0)),
                   pl.BlockSpec((1, 20, 1), lambda g: (g, 0, 0))],
        out_shape=[jax.ShapeDtypeStruct((B, 20, 1), F32),
                   jax.ShapeDtypeStruct((B, 20, 1), F32)],
    )(*args)


def kernel(x, edge_index, edge_attr, state, batch, bond_batch, params):
    src = edge_index[0]
    dst = edge_index[1]
    bbrow = bond_batch.astype(F32).reshape(GE, 1, BE)
    bbcol = bond_batch.astype(F32).reshape(GE, BE, 1)
    brow = batch.astype(F32).reshape(GN, 1, BN)
    bcol = batch.astype(F32).reshape(GN, BN, 1)

    xv = x.astype(F32)
    ev = edge_attr
    uv = state
    cnt_n32 = None
    uxm = uem = None
    for bi, p in enumerate(params["blocks"]):
        first = bi == 0
        x0, u0 = _pre_call(xv, uv, p["npre"], p["spre"])
        gx, gy = _sc_gather(x0, src, dst)
        e1, ue_sum, cnt_e = _edge_call(ev, gx, gy, bbrow, bbcol, u0, p, first)
        if first:
            psum, pcnt = _sc_scatter(e1, dst, True)
            x1, u1, uxm, _, cnt_n32 = _node_call(
                psum, pcnt, x0, x0, brow, bcol, u0, u0, ue_sum, cnt_e,
                p, True)
        else:
            psum = _sc_scatter(e1, dst, False)
            x1, u1, uxm, _ = _node_call(
                psum, cnt_n32, x0, xv, brow, bcol, u0, uv, ue_sum, cnt_e,
                p, False)
        uem = ue_sum / jnp.maximum(cnt_e, 1.0)
        xv, ev, uv = x1, e1, u1

    su_w, sd_w, fermi = _head_rnn_call(uxm, uem, uv, params)
    up, dn = _attn_call(su_w.reshape(B, 20, 128), sd_w.reshape(B, 20, 128),
                        params)
    hu = up[..., 0]
    hd = dn[..., 0]
    x_uu = hu[:, 10:]
    x_ud = hd[:, 10:]
    x_du = jnp.flip(hu[:, :10], axis=1)
    x_dd = jnp.flip(hd[:, :10], axis=1)
    return (x_uu, x_ud, x_du, x_dd, fermi)
